# baseline (speedup 1.0000x reference)
import functools

import jax
import jax.numpy as jnp
from jax import lax
from jax.experimental import pallas as pl
from jax.experimental.pallas import tpu as pltpu
from jax.experimental.pallas import tpu_sc as plsc

NC = 2
NS = 16
NW = NC * NS

CHUNK = 128
TILE = 128
DPAD = 384


def _gather_body(idx_hbm, out_hbm, w_hbm, wt_hbm, idx_v, buf0, buf1, gsems,
                 wsems, wid, nchunks):
    base = pl.multiple_of(wid * nchunks * CHUNK, CHUNK)

    pltpu.sync_copy(idx_hbm.at[wid], idx_v)

    def start_gathers(c, buf, sem):
        idx = idx_v.at[c]
        g0 = pltpu.async_copy(w_hbm.at[idx, pl.ds(0, TILE)],
                              buf.at[:, pl.ds(0, TILE)], sem)
        g1 = pltpu.async_copy(w_hbm.at[idx, pl.ds(TILE, TILE)],
                              buf.at[:, pl.ds(TILE, TILE)], sem)
        g2 = pltpu.async_copy(wt_hbm.at[idx],
                              buf.at[:, pl.ds(2 * TILE, TILE)], sem)
        return (g0, g1, g2)

    def step(i, _):
        c0 = pl.multiple_of(2 * i * CHUNK, CHUNK)
        c1 = pl.multiple_of((2 * i + 1) * CHUNK, CHUNK)
        ga = start_gathers(2 * i, buf0, gsems[0])
        gb = start_gathers(2 * i + 1, buf1, gsems[1])
        for g in ga:
            g.wait()
        w0 = pltpu.async_copy(buf0, out_hbm.at[pl.ds(base + c0, CHUNK)],
                              wsems[0])
        for g in gb:
            g.wait()
        w1 = pltpu.async_copy(buf1, out_hbm.at[pl.ds(base + c1, CHUNK)],
                              wsems[1])
        w0.wait()
        w1.wait()
        return 0

    lax.fori_loop(0, nchunks // 2, step, 0)


def _make_sc_gather(n_total):
    n_per_w = n_total // NW
    nchunks = n_per_w // CHUNK
    mesh = plsc.VectorSubcoreMesh(core_axis_name="c", subcore_axis_name="s")

    @functools.partial(
        pl.kernel,
        out_type=jax.ShapeDtypeStruct((n_total, DPAD), jnp.float32),
        mesh=mesh,
        scratch_types=[
            pltpu.VMEM((nchunks, CHUNK), jnp.int32),
            pltpu.VMEM((CHUNK, DPAD), jnp.float32),
            pltpu.VMEM((CHUNK, DPAD), jnp.float32),
            pltpu.SemaphoreType.DMA,
            pltpu.SemaphoreType.DMA,
            pltpu.SemaphoreType.DMA,
            pltpu.SemaphoreType.DMA,
        ],
    )
    def sc_gather(idx_hbm, w_hbm, wt_hbm, out_hbm, idx_v, buf0, buf1,
                  g0, g1, w0, w1):
        wid = lax.axis_index("s") * NC + lax.axis_index("c")
        _gather_body(idx_hbm, out_hbm, w_hbm, wt_hbm, idx_v, buf0, buf1,
                     (g0, g1), (w0, w1), wid, nchunks)

    return sc_gather


def _slice_out(o_pad, b, l, dim):
    def body(i_ref, o_ref):
        o_ref[0] = i_ref[:l, :dim]
        o_ref[1] = i_ref[l:, :dim]

    return pl.pallas_call(
        body,
        grid=(b // 2,),
        in_specs=[pl.BlockSpec((2 * l, DPAD), lambda i: (i, 0))],
        out_specs=pl.BlockSpec((2, l, dim), lambda i: (i, 0, 0)),
        out_shape=jax.ShapeDtypeStruct((b, l, dim), jnp.float32),
    )(o_pad)


def kernel(target_word, synonym, antonym, W):
    b, l = target_word.shape
    dim = W.shape[1]
    n = b * l
    nchunks = n // NW // CHUNK
    w_tail = jnp.pad(W[:, 2 * TILE:], ((0, 0), (0, 3 * TILE - dim)))

    fn = _make_sc_gather(n)
    outs = []
    for idx in (target_word, synonym, antonym):
        idx3 = idx.reshape(NW, nchunks, CHUNK).astype(jnp.int32)
        outs.append(_slice_out(fn(idx3, W, w_tail), b, l, dim))
    return tuple(outs)


# speedup vs baseline: 3.8451x; 3.8451x over previous
import functools

import jax
import jax.numpy as jnp
from jax import lax
from jax.experimental import pallas as pl
from jax.experimental.pallas import tpu as pltpu
from jax.experimental.pallas import tpu_sc as plsc

NC = 2
NS = 16
NW = NC * NS

CHUNK = 128
TILE = 128
DPAD = 384


def _gather_body(idx_hbm, out_hbm, w_hbm, wt_hbm, idx_v, buf0, buf1, gsems,
                 wsems, wid, nchunks):
    base = pl.multiple_of(wid * nchunks * CHUNK, CHUNK)

    pltpu.sync_copy(idx_hbm.at[wid], idx_v)

    def start_gathers(c, buf, sem):
        idx = idx_v.at[c]
        g0 = pltpu.async_copy(w_hbm.at[idx, pl.ds(0, TILE)],
                              buf.at[:, pl.ds(0, TILE)], sem)
        g1 = pltpu.async_copy(w_hbm.at[idx, pl.ds(TILE, TILE)],
                              buf.at[:, pl.ds(TILE, TILE)], sem)
        g2 = pltpu.async_copy(wt_hbm.at[idx],
                              buf.at[:, pl.ds(2 * TILE, TILE)], sem)
        return (g0, g1, g2)

    def step(i, _):
        c0 = pl.multiple_of(2 * i * CHUNK, CHUNK)
        c1 = pl.multiple_of((2 * i + 1) * CHUNK, CHUNK)
        ga = start_gathers(2 * i, buf0, gsems[0])
        gb = start_gathers(2 * i + 1, buf1, gsems[1])
        for g in ga:
            g.wait()
        w0 = pltpu.async_copy(buf0, out_hbm.at[pl.ds(base + c0, CHUNK)],
                              wsems[0])
        for g in gb:
            g.wait()
        w1 = pltpu.async_copy(buf1, out_hbm.at[pl.ds(base + c1, CHUNK)],
                              wsems[1])
        w0.wait()
        w1.wait()
        return 0

    lax.fori_loop(0, nchunks // 2, step, 0)


def _make_sc_gather(n_total):
    n_per_w = n_total // NW
    nchunks = n_per_w // CHUNK
    mesh = plsc.VectorSubcoreMesh(core_axis_name="c", subcore_axis_name="s")

    @functools.partial(
        pl.kernel,
        out_type=jax.ShapeDtypeStruct((n_total, DPAD), jnp.float32),
        mesh=mesh,
        scratch_types=[
            pltpu.VMEM((nchunks, CHUNK), jnp.int32),
            pltpu.VMEM((CHUNK, DPAD), jnp.float32),
            pltpu.VMEM((CHUNK, DPAD), jnp.float32),
            pltpu.SemaphoreType.DMA,
            pltpu.SemaphoreType.DMA,
            pltpu.SemaphoreType.DMA,
            pltpu.SemaphoreType.DMA,
        ],
    )
    def sc_gather(idx_hbm, w_hbm, wt_hbm, out_hbm, idx_v, buf0, buf1,
                  g0, g1, w0, w1):
        wid = lax.axis_index("s") * NC + lax.axis_index("c")
        _gather_body(idx_hbm, out_hbm, w_hbm, wt_hbm, idx_v, buf0, buf1,
                     (g0, g1), (w0, w1), wid, nchunks)

    return sc_gather


def kernel(target_word, synonym, antonym, W):
    b, l = target_word.shape
    dim = W.shape[1]
    n = b * l
    nchunks = n // NW // CHUNK
    w_tail = jnp.pad(W[:, 2 * TILE:], ((0, 0), (0, 3 * TILE - dim)))

    fn = _make_sc_gather(n)
    outs = []
    for idx in (target_word, synonym, antonym):
        idx3 = idx.reshape(NW, nchunks, CHUNK).astype(jnp.int32)
        outs.append(fn(idx3, W, w_tail)[:, :dim].reshape(b, l, dim))
    return tuple(outs)


# speedup vs baseline: 3.8467x; 1.0004x over previous
import functools

import jax
import jax.numpy as jnp
from jax import lax
from jax.experimental import pallas as pl
from jax.experimental.pallas import tpu as pltpu
from jax.experimental.pallas import tpu_sc as plsc

NC = 2
NS = 16
NW = NC * NS

CHUNK = 64
NBUF = 4
TILE = 128
DPAD = 384


def _gather_body(idx_hbm, out_hbm, w_hbm, wt_hbm, idx_v, bufs, gsems,
                 wsems, wid, nchunks):
    base = pl.multiple_of(wid * nchunks * CHUNK, CHUNK)

    pltpu.sync_copy(idx_hbm.at[wid], idx_v)

    def start_gathers(c, buf, sem):
        idx = idx_v.at[c]
        g0 = pltpu.async_copy(w_hbm.at[idx, pl.ds(0, TILE)],
                              buf.at[:, pl.ds(0, TILE)], sem)
        g1 = pltpu.async_copy(w_hbm.at[idx, pl.ds(TILE, TILE)],
                              buf.at[:, pl.ds(TILE, TILE)], sem)
        g2 = pltpu.async_copy(wt_hbm.at[idx],
                              buf.at[:, pl.ds(2 * TILE, TILE)], sem)
        return (g0, g1, g2)

    def step(i, _):
        gs = [start_gathers(NBUF * i + j, bufs[j], gsems[j])
              for j in range(NBUF)]
        ws = []
        for j in range(NBUF):
            for g in gs[j]:
                g.wait()
            off = pl.multiple_of((NBUF * i + j) * CHUNK, CHUNK)
            ws.append(pltpu.async_copy(
                bufs[j], out_hbm.at[pl.ds(base + off, CHUNK)], wsems[j]))
        for w in ws:
            w.wait()
        return 0

    lax.fori_loop(0, nchunks // NBUF, step, 0)


def _make_sc_gather(n_total):
    n_per_w = n_total // NW
    nchunks = n_per_w // CHUNK
    mesh = plsc.VectorSubcoreMesh(core_axis_name="c", subcore_axis_name="s")

    @functools.partial(
        pl.kernel,
        out_type=jax.ShapeDtypeStruct((n_total, DPAD), jnp.float32),
        mesh=mesh,
        scratch_types=(
            [pltpu.VMEM((nchunks, CHUNK), jnp.int32)]
            + [pltpu.VMEM((CHUNK, DPAD), jnp.float32)] * NBUF
            + [pltpu.SemaphoreType.DMA] * (2 * NBUF)
        ),
    )
    def sc_gather(idx_hbm, w_hbm, wt_hbm, out_hbm, idx_v, *rest):
        bufs = rest[:NBUF]
        gsems = rest[NBUF:2 * NBUF]
        wsems = rest[2 * NBUF:3 * NBUF]
        wid = lax.axis_index("s") * NC + lax.axis_index("c")
        _gather_body(idx_hbm, out_hbm, w_hbm, wt_hbm, idx_v, bufs,
                     gsems, wsems, wid, nchunks)

    return sc_gather


def kernel(target_word, synonym, antonym, W):
    b, l = target_word.shape
    dim = W.shape[1]
    n = b * l
    nchunks = n // NW // CHUNK
    w_tail = jnp.pad(W[:, 2 * TILE:], ((0, 0), (0, 3 * TILE - dim)))

    fn = _make_sc_gather(n)
    outs = []
    for idx in (target_word, synonym, antonym):
        idx3 = idx.reshape(NW, nchunks, CHUNK).astype(jnp.int32)
        outs.append(fn(idx3, W, w_tail)[:, :dim].reshape(b, l, dim))
    return tuple(outs)
